# baseline (speedup 1.0000x reference)
import functools

import jax
import jax.numpy as jnp
from jax import lax
from jax.experimental import pallas as pl
from jax.experimental.pallas import tpu as pltpu
from jax.experimental.pallas import tpu_sc as plsc

N = 10000
E = 320000
D_IN = 128
H = 32
OUT = 32
G = 64

NC, NS = 2, 16
NW = NC * NS
NPAD = 10240
EPAD = 327680
EPW = EPAD // NW
SLAB = 128
SLABS_PW = EPW // SLAB
CS = 8
NCHUNK = SLABS_PW // CS
F0 = 48
F1 = 112
RPT = NPAD // NS
BM = 1024
GRID = NPAD // BM

_sc_mesh = plsc.VectorSubcoreMesh(
    core_axis_name="c", subcore_axis_name="s", num_cores=NC, num_subcores=NS)
_sc_params = pltpu.CompilerParams(use_tc_tiling_on_sc=False)


@functools.partial(
    pl.kernel,
    out_type=jax.ShapeDtypeStruct((NC, NPAD), jnp.float32),
    mesh=_sc_mesh,
    compiler_params=_sc_params,
    scratch_types=[
        pltpu.VMEM_SHARED((NPAD,), jnp.float32),
        pltpu.VMEM((CS, SLAB), jnp.int32),
        pltpu.VMEM((SLAB,), jnp.float32),
        pltpu.VMEM((RPT,), jnp.float32),
    ],
)
def _deg_sc(dst_hbm, out_hbm, acc, idx_v, ones_v, buf_v):
    c = lax.axis_index("c")
    s = lax.axis_index("s")
    wid = s * NC + c

    def _zero(i, _):
        buf_v[pl.ds(i * 16, 16)] = jnp.zeros((16,), jnp.float32)
        return 0
    lax.fori_loop(0, RPT // 16, _zero, 0)
    pltpu.sync_copy(buf_v, acc.at[pl.ds(s * RPT, RPT)])

    def _ones(i, _):
        ones_v[pl.ds(i * 16, 16)] = jnp.ones((16,), jnp.float32)
        return 0
    lax.fori_loop(0, SLAB // 16, _ones, 0)
    plsc.subcore_barrier()

    def _chunk(g, _):
        slab0 = wid * SLABS_PW + g * CS
        pltpu.sync_copy(dst_hbm.at[pl.ds(slab0, CS)], idx_v)
        for j in range(CS):
            pltpu.sync_copy(ones_v, acc.at[idx_v.at[j]], add=True)
        return 0
    lax.fori_loop(0, NCHUNK, _chunk, 0)

    plsc.subcore_barrier()
    pltpu.sync_copy(acc.at[pl.ds(s * RPT, RPT)], buf_v)
    pltpu.sync_copy(buf_v, out_hbm.at[c, pl.ds(s * RPT, RPT)])


@functools.partial(
    pl.kernel,
    out_type=jax.ShapeDtypeStruct((NC, NPAD, H), jnp.float32),
    mesh=_sc_mesh,
    compiler_params=_sc_params,
    scratch_types=[
        pltpu.VMEM_SHARED((NPAD, H), jnp.float32),
        pltpu.VMEM((CS, SLAB), jnp.int32),
        pltpu.VMEM((CS, SLAB), jnp.int32),
        pltpu.VMEM((CS * SLAB, H), jnp.float32),
        pltpu.SemaphoreType.DMA,
        pltpu.SemaphoreType.DMA,
    ],
)
def _prop_sc(y_hbm, src_hbm, dst_hbm, out_hbm, acc, idx_s, idx_d, rows_v,
             sem, sem_s):
    c = lax.axis_index("c")
    s = lax.axis_index("s")
    wid = s * NC + c
    rbase = s * RPT

    def _zero(i, _):
        rows_v[i, pl.ds(0, 16)] = jnp.zeros((16,), jnp.float32)
        rows_v[i, pl.ds(16, 16)] = jnp.zeros((16,), jnp.float32)
        return 0
    lax.fori_loop(0, RPT, _zero, 0)

    pltpu.sync_copy(rows_v.at[pl.ds(0, RPT)], acc.at[pl.ds(rbase, RPT)])
    plsc.subcore_barrier()

    base = jnp.where(c == 0, s * F0, 16 * F0 + s * F1)
    nch = jnp.where(c == 0, F0 // CS, F1 // CS)

    def _chunk(g, _):
        slab0 = base + g * CS
        pltpu.sync_copy(src_hbm.at[pl.ds(slab0, CS)], idx_s)
        pltpu.sync_copy(dst_hbm.at[pl.ds(slab0, CS)], idx_d)
        cps = [
            pltpu.async_copy(
                y_hbm.at[idx_s.at[j]], rows_v.at[pl.ds(j * SLAB, SLAB)], sem)
            for j in range(CS)
        ]
        sps = []
        for j in range(CS):
            cps[j].wait()
            sps.append(pltpu.async_copy(
                rows_v.at[pl.ds(j * SLAB, SLAB)], acc.at[idx_d.at[j]], sem_s,
                add=True))
        for sp in sps:
            sp.wait()
        return 0
    lax.fori_loop(0, nch, _chunk, 0)

    plsc.subcore_barrier()
    pltpu.sync_copy(acc.at[pl.ds(rbase, RPT)], rows_v.at[pl.ds(0, RPT)])
    pltpu.sync_copy(rows_v.at[pl.ds(0, RPT)], out_hbm.at[c, pl.ds(rbase, RPT)])


def _k1_body(x_ref, w_ref, degp_ref, y_ref, dinv_ref):
    i = pl.program_id(0)
    deg = degp_ref[0] + degp_ref[1] + 1.0
    rows = i * BM + lax.broadcasted_iota(jnp.int32, (BM, 1), 0)
    dinv = jnp.where(rows < N, lax.rsqrt(jnp.maximum(deg, 1.0)), 0.0)
    xw = jnp.dot(x_ref[...], w_ref[...], preferred_element_type=jnp.float32)
    y_ref[...] = xw * dinv
    dinv_ref[...] = dinv


def _k1_call(xp, W1, degp3):
    return pl.pallas_call(
        _k1_body,
        grid=(GRID,),
        in_specs=[
            pl.BlockSpec((BM, D_IN), lambda i: (i, 0)),
            pl.BlockSpec((D_IN, H), lambda i: (0, 0)),
            pl.BlockSpec((NC, BM, 1), lambda i: (0, i, 0)),
        ],
        out_specs=[
            pl.BlockSpec((BM, H), lambda i: (i, 0)),
            pl.BlockSpec((BM, 1), lambda i: (i, 0)),
        ],
        out_shape=[
            jax.ShapeDtypeStruct((NPAD, H), jnp.float32),
            jax.ShapeDtypeStruct((NPAD, 1), jnp.float32),
        ],
    )(xp, W1, degp3)


def _stats_body(p_ref, y_ref, dinv_ref, out_ref):
    i = pl.program_id(0)
    t = (p_ref[0] + p_ref[1] + y_ref[...]) * dinv_ref[...]
    st = jnp.concatenate(
        [jnp.sum(t, axis=0, keepdims=True),
         jnp.sum(t * t, axis=0, keepdims=True)], axis=0)

    @pl.when(i == 0)
    def _():
        out_ref[...] = st

    @pl.when(i > 0)
    def _():
        out_ref[...] += st


def _stats_call(p, y, dinv):
    return pl.pallas_call(
        _stats_body,
        grid=(GRID,),
        in_specs=[
            pl.BlockSpec((NC, BM, H), lambda i: (0, i, 0)),
            pl.BlockSpec((BM, H), lambda i: (i, 0)),
            pl.BlockSpec((BM, 1), lambda i: (i, 0)),
        ],
        out_specs=pl.BlockSpec((2, H), lambda i: (0, 0)),
        out_shape=jax.ShapeDtypeStruct((2, H), jnp.float32),
    )(p, y, dinv)


def _rc_body(p_ref, yp_ref, dinv_ref, ac_ref, w_ref, y_ref):
    dinv = dinv_ref[...]
    t = (p_ref[0] + p_ref[1] + yp_ref[...]) * dinv
    z = jnp.maximum(t * ac_ref[0:1, :] + ac_ref[1:2, :], 0.0)
    y_ref[...] = jnp.dot(
        z, w_ref[...], preferred_element_type=jnp.float32) * dinv


def _rc_call(p, yp, dinv, ac, W):
    return pl.pallas_call(
        _rc_body,
        grid=(GRID,),
        in_specs=[
            pl.BlockSpec((NC, BM, H), lambda i: (0, i, 0)),
            pl.BlockSpec((BM, H), lambda i: (i, 0)),
            pl.BlockSpec((BM, 1), lambda i: (i, 0)),
            pl.BlockSpec((2, H), lambda i: (0, 0)),
            pl.BlockSpec((H, H), lambda i: (0, 0)),
        ],
        out_specs=pl.BlockSpec((BM, H), lambda i: (i, 0)),
        out_shape=jax.ShapeDtypeStruct((NPAD, H), jnp.float32),
    )(p, yp, dinv, ac, W)


def _pool_body(p_ref, yp_ref, dinv_ref, b3_ref, batch_ref, out_ref, sums_v,
               cnt_v):
    i = pl.program_id(0)

    @pl.when(i == 0)
    def _():
        sums_v[...] = jnp.zeros_like(sums_v)
        cnt_v[...] = jnp.zeros_like(cnt_v)

    t = (p_ref[0] + p_ref[1] + yp_ref[...]) * dinv_ref[...]
    h = t + b3_ref[...]
    b = batch_ref[0]
    oh = (lax.broadcasted_iota(jnp.int32, (G, BM), 0) == b
          ).astype(jnp.float32)
    sums_v[...] += jnp.dot(oh, h, preferred_element_type=jnp.float32)
    cnt_v[...] += jnp.sum(oh, axis=1, keepdims=True)
    out_ref[...] = sums_v[...] / jnp.maximum(cnt_v[...], 1.0)


def _pool_call(p, yp, dinv, b3, batchp):
    return pl.pallas_call(
        _pool_body,
        grid=(GRID,),
        in_specs=[
            pl.BlockSpec((NC, BM, H), lambda i: (0, i, 0)),
            pl.BlockSpec((BM, H), lambda i: (i, 0)),
            pl.BlockSpec((BM, 1), lambda i: (i, 0)),
            pl.BlockSpec((1, H), lambda i: (0, 0)),
            pl.BlockSpec((1, 1, BM), lambda i: (i, 0, 0)),
        ],
        out_specs=pl.BlockSpec((G, OUT), lambda i: (0, 0)),
        out_shape=jax.ShapeDtypeStruct((G, OUT), jnp.float32),
        scratch_shapes=[
            pltpu.VMEM((G, OUT), jnp.float32),
            pltpu.VMEM((G, 1), jnp.float32),
        ],
    )(p, yp, dinv, b3, batchp)


def kernel(x, edge_index, batch, W1, b1, W2, b2, W3, b3,
           gamma1, beta1, gamma2, beta2):
    x = x.astype(jnp.float32)
    src = edge_index[0].astype(jnp.int32)
    dst = edge_index[1].astype(jnp.int32)
    srcp = jnp.concatenate(
        [src, jnp.full((EPAD - E,), N, jnp.int32)]).reshape(EPAD // SLAB, SLAB)
    dstp = jnp.concatenate(
        [dst, jnp.full((EPAD - E,), NPAD - 1, jnp.int32)]
    ).reshape(EPAD // SLAB, SLAB)
    xp = jnp.pad(x, ((0, NPAD - N), (0, 0)))
    batchp = jnp.concatenate(
        [batch.astype(jnp.int32), jnp.full((NPAD - N,), G, jnp.int32)]
    ).reshape(GRID, 1, BM)

    degp = _deg_sc(dstp)
    y1, dinv = _k1_call(xp, W1, degp.reshape(NC, NPAD, 1))

    p1 = _prop_sc(y1, srcp, dstp)
    s1 = _stats_call(p1, y1, dinv)
    m1 = s1[0] / N
    a1 = gamma1 * lax.rsqrt(s1[1] / N - m1 * m1 + 1e-5)
    c1 = beta1 - m1 * a1
    y2 = _rc_call(p1, y1, dinv, jnp.stack([a1, c1]), W2)

    p2 = _prop_sc(y2, srcp, dstp)
    s2 = _stats_call(p2, y2, dinv)
    m2 = s2[0] / N
    a2 = gamma2 * lax.rsqrt(s2[1] / N - m2 * m2 + 1e-5)
    c2 = beta2 - m2 * a2
    y3 = _rc_call(p2, y2, dinv, jnp.stack([a2, c2]), W3)

    p3 = _prop_sc(y3, srcp, dstp)
    return _pool_call(p3, y3, dinv, b3.reshape(1, OUT), batchp)


# speedup vs baseline: 1.7297x; 1.7297x over previous
import functools

import jax
import jax.numpy as jnp
from jax import lax
from jax.experimental import pallas as pl
from jax.experimental.pallas import tpu as pltpu
from jax.experimental.pallas import tpu_sc as plsc

N = 10000
E = 320000
D_IN = 128
H = 32
OUT = 32
G = 64

NC, NS = 2, 16
NW = NC * NS
NPAD = 10240
EPAD = 327680
EPW = EPAD // NW
SLAB = 128
SLABS_PW = EPW // SLAB
CS = 8
NCHUNK = SLABS_PW // CS
RPT = NPAD // NS
BM = 1024
GRID = NPAD // BM

_sc_mesh = plsc.VectorSubcoreMesh(
    core_axis_name="c", subcore_axis_name="s", num_cores=NC, num_subcores=NS)
_sc_params = pltpu.CompilerParams(use_tc_tiling_on_sc=False)


@functools.partial(
    pl.kernel,
    out_type=jax.ShapeDtypeStruct((NC, NPAD), jnp.float32),
    mesh=_sc_mesh,
    compiler_params=_sc_params,
    scratch_types=[
        pltpu.VMEM_SHARED((NPAD,), jnp.float32),
        pltpu.VMEM((CS, SLAB), jnp.int32),
        pltpu.VMEM((SLAB,), jnp.float32),
        pltpu.VMEM((RPT,), jnp.float32),
    ],
)
def _deg_sc(dst_hbm, out_hbm, acc, idx_v, ones_v, buf_v):
    c = lax.axis_index("c")
    s = lax.axis_index("s")
    wid = s * NC + c

    def _zero(i, _):
        buf_v[pl.ds(i * 16, 16)] = jnp.zeros((16,), jnp.float32)
        return 0
    lax.fori_loop(0, RPT // 16, _zero, 0)
    pltpu.sync_copy(buf_v, acc.at[pl.ds(s * RPT, RPT)])

    def _ones(i, _):
        ones_v[pl.ds(i * 16, 16)] = jnp.ones((16,), jnp.float32)
        return 0
    lax.fori_loop(0, SLAB // 16, _ones, 0)
    plsc.subcore_barrier()

    def _chunk(g, _):
        slab0 = wid * SLABS_PW + g * CS
        pltpu.sync_copy(dst_hbm.at[pl.ds(slab0, CS)], idx_v)
        for j in range(CS):
            pltpu.sync_copy(ones_v, acc.at[idx_v.at[j]], add=True)
        return 0
    lax.fori_loop(0, NCHUNK, _chunk, 0)

    plsc.subcore_barrier()
    pltpu.sync_copy(acc.at[pl.ds(s * RPT, RPT)], buf_v)
    pltpu.sync_copy(buf_v, out_hbm.at[c, pl.ds(s * RPT, RPT)])


@functools.partial(
    pl.kernel,
    out_type=jax.ShapeDtypeStruct((NC, NPAD, H), jnp.float32),
    mesh=_sc_mesh,
    compiler_params=_sc_params,
    scratch_types=[
        pltpu.VMEM_SHARED((NPAD, H), jnp.float32),
        pltpu.VMEM_SHARED((NPAD, H), jnp.float32),
        pltpu.VMEM((CS, SLAB), jnp.int32),
        pltpu.VMEM((CS, SLAB), jnp.int32),
        pltpu.VMEM((CS * SLAB, H), jnp.float32),
        pltpu.SemaphoreType.DMA,
        pltpu.SemaphoreType.DMA,
    ],
)
def _prop_sc(y_hbm, src_hbm, dst_hbm, out_hbm, acc, y_s, idx_s, idx_d, rows_v,
             sem, sem_s):
    c = lax.axis_index("c")
    s = lax.axis_index("s")
    wid = s * NC + c
    rbase = s * RPT

    pltpu.sync_copy(y_hbm.at[pl.ds(rbase, RPT)], rows_v.at[pl.ds(0, RPT)])
    pltpu.sync_copy(rows_v.at[pl.ds(0, RPT)], y_s.at[pl.ds(rbase, RPT)])

    @pl.when(c == 0)
    def _():
        pltpu.sync_copy(rows_v.at[pl.ds(0, RPT)], acc.at[pl.ds(rbase, RPT)])

    @pl.when(c != 0)
    def _():
        def _zero(i, _):
            rows_v[i, pl.ds(0, 16)] = jnp.zeros((16,), jnp.float32)
            rows_v[i, pl.ds(16, 16)] = jnp.zeros((16,), jnp.float32)
            return 0
        lax.fori_loop(0, RPT, _zero, 0)
        pltpu.sync_copy(rows_v.at[pl.ds(0, RPT)], acc.at[pl.ds(rbase, RPT)])

    plsc.subcore_barrier()

    def _chunk(g, _):
        slab0 = wid * SLABS_PW + g * CS
        pltpu.sync_copy(src_hbm.at[pl.ds(slab0, CS)], idx_s)
        pltpu.sync_copy(dst_hbm.at[pl.ds(slab0, CS)], idx_d)
        cps = [
            pltpu.async_copy(
                y_s.at[idx_s.at[j]], rows_v.at[pl.ds(j * SLAB, SLAB)], sem)
            for j in range(CS)
        ]
        sps = []
        for j in range(CS):
            cps[j].wait()
            sps.append(pltpu.async_copy(
                rows_v.at[pl.ds(j * SLAB, SLAB)], acc.at[idx_d.at[j]], sem_s,
                add=True))
        for sp in sps:
            sp.wait()
        return 0
    lax.fori_loop(0, NCHUNK, _chunk, 0)

    plsc.subcore_barrier()
    pltpu.sync_copy(acc.at[pl.ds(rbase, RPT)], rows_v.at[pl.ds(0, RPT)])
    pltpu.sync_copy(rows_v.at[pl.ds(0, RPT)], out_hbm.at[c, pl.ds(rbase, RPT)])


def _k1_body(x_ref, w_ref, degp_ref, y_ref, dinv_ref):
    i = pl.program_id(0)
    deg = degp_ref[0] + degp_ref[1] + 1.0
    rows = i * BM + lax.broadcasted_iota(jnp.int32, (BM, 1), 0)
    dinv = jnp.where(rows < N, lax.rsqrt(jnp.maximum(deg, 1.0)), 0.0)
    xw = jnp.dot(x_ref[...], w_ref[...], preferred_element_type=jnp.float32)
    y_ref[...] = xw * dinv
    dinv_ref[...] = dinv


def _k1_call(xp, W1, degp3):
    return pl.pallas_call(
        _k1_body,
        grid=(GRID,),
        in_specs=[
            pl.BlockSpec((BM, D_IN), lambda i: (i, 0)),
            pl.BlockSpec((D_IN, H), lambda i: (0, 0)),
            pl.BlockSpec((NC, BM, 1), lambda i: (0, i, 0)),
        ],
        out_specs=[
            pl.BlockSpec((BM, H), lambda i: (i, 0)),
            pl.BlockSpec((BM, 1), lambda i: (i, 0)),
        ],
        out_shape=[
            jax.ShapeDtypeStruct((NPAD, H), jnp.float32),
            jax.ShapeDtypeStruct((NPAD, 1), jnp.float32),
        ],
    )(xp, W1, degp3)


def _stats_body(p_ref, dinv_ref, out_ref):
    i = pl.program_id(0)
    t = (p_ref[0] + p_ref[1]) * dinv_ref[...]
    st = jnp.concatenate(
        [jnp.sum(t, axis=0, keepdims=True),
         jnp.sum(t * t, axis=0, keepdims=True)], axis=0)

    @pl.when(i == 0)
    def _():
        out_ref[...] = st

    @pl.when(i > 0)
    def _():
        out_ref[...] += st


def _stats_call(p, dinv):
    return pl.pallas_call(
        _stats_body,
        grid=(GRID,),
        in_specs=[
            pl.BlockSpec((NC, BM, H), lambda i: (0, i, 0)),
            pl.BlockSpec((BM, 1), lambda i: (i, 0)),
        ],
        out_specs=pl.BlockSpec((2, H), lambda i: (0, 0)),
        out_shape=jax.ShapeDtypeStruct((2, H), jnp.float32),
    )(p, dinv)


def _rc_body(p_ref, dinv_ref, ac_ref, w_ref, y_ref):
    dinv = dinv_ref[...]
    t = (p_ref[0] + p_ref[1]) * dinv
    z = jnp.maximum(t * ac_ref[0:1, :] + ac_ref[1:2, :], 0.0)
    y_ref[...] = jnp.dot(
        z, w_ref[...], preferred_element_type=jnp.float32) * dinv


def _rc_call(p, dinv, ac, W):
    return pl.pallas_call(
        _rc_body,
        grid=(GRID,),
        in_specs=[
            pl.BlockSpec((NC, BM, H), lambda i: (0, i, 0)),
            pl.BlockSpec((BM, 1), lambda i: (i, 0)),
            pl.BlockSpec((2, H), lambda i: (0, 0)),
            pl.BlockSpec((H, H), lambda i: (0, 0)),
        ],
        out_specs=pl.BlockSpec((BM, H), lambda i: (i, 0)),
        out_shape=jax.ShapeDtypeStruct((NPAD, H), jnp.float32),
    )(p, dinv, ac, W)


def _pool_body(p_ref, dinv_ref, b3_ref, batch_ref, out_ref, sums_v, cnt_v):
    i = pl.program_id(0)

    @pl.when(i == 0)
    def _():
        sums_v[...] = jnp.zeros_like(sums_v)
        cnt_v[...] = jnp.zeros_like(cnt_v)

    t = (p_ref[0] + p_ref[1]) * dinv_ref[...]
    h = t + b3_ref[...]
    b = batch_ref[0]
    oh = (lax.broadcasted_iota(jnp.int32, (G, BM), 0) == b
          ).astype(jnp.float32)
    sums_v[...] += jnp.dot(oh, h, preferred_element_type=jnp.float32)
    cnt_v[...] += jnp.sum(oh, axis=1, keepdims=True)
    out_ref[...] = sums_v[...] / jnp.maximum(cnt_v[...], 1.0)


def _pool_call(p, dinv, b3, batchp):
    return pl.pallas_call(
        _pool_body,
        grid=(GRID,),
        in_specs=[
            pl.BlockSpec((NC, BM, H), lambda i: (0, i, 0)),
            pl.BlockSpec((BM, 1), lambda i: (i, 0)),
            pl.BlockSpec((1, H), lambda i: (0, 0)),
            pl.BlockSpec((1, 1, BM), lambda i: (i, 0, 0)),
        ],
        out_specs=pl.BlockSpec((G, OUT), lambda i: (0, 0)),
        out_shape=jax.ShapeDtypeStruct((G, OUT), jnp.float32),
        scratch_shapes=[
            pltpu.VMEM((G, OUT), jnp.float32),
            pltpu.VMEM((G, 1), jnp.float32),
        ],
    )(p, dinv, b3, batchp)


def kernel(x, edge_index, batch, W1, b1, W2, b2, W3, b3,
           gamma1, beta1, gamma2, beta2):
    x = x.astype(jnp.float32)
    src = edge_index[0].astype(jnp.int32)
    dst = edge_index[1].astype(jnp.int32)
    srcp = jnp.concatenate(
        [src, jnp.full((EPAD - E,), N, jnp.int32)]).reshape(EPAD // SLAB, SLAB)
    dstp = jnp.concatenate(
        [dst, jnp.full((EPAD - E,), NPAD - 1, jnp.int32)]
    ).reshape(EPAD // SLAB, SLAB)
    xp = jnp.pad(x, ((0, NPAD - N), (0, 0)))
    batchp = jnp.concatenate(
        [batch.astype(jnp.int32), jnp.full((NPAD - N,), G, jnp.int32)]
    ).reshape(GRID, 1, BM)

    degp = _deg_sc(dstp)
    y1, dinv = _k1_call(xp, W1, degp.reshape(NC, NPAD, 1))

    p1 = _prop_sc(y1, srcp, dstp)
    s1 = _stats_call(p1, dinv)
    m1 = s1[0] / N
    a1 = gamma1 * lax.rsqrt(s1[1] / N - m1 * m1 + 1e-5)
    c1 = beta1 - m1 * a1
    y2 = _rc_call(p1, dinv, jnp.stack([a1, c1]), W2)

    p2 = _prop_sc(y2, srcp, dstp)
    s2 = _stats_call(p2, dinv)
    m2 = s2[0] / N
    a2 = gamma2 * lax.rsqrt(s2[1] / N - m2 * m2 + 1e-5)
    c2 = beta2 - m2 * a2
    y3 = _rc_call(p2, dinv, jnp.stack([a2, c2]), W3)

    p3 = _prop_sc(y3, srcp, dstp)
    return _pool_call(p3, dinv, b3.reshape(1, OUT), batchp)


# speedup vs baseline: 1.7316x; 1.0011x over previous
import functools

import jax
import jax.numpy as jnp
from jax import lax
from jax.experimental import pallas as pl
from jax.experimental.pallas import tpu as pltpu
from jax.experimental.pallas import tpu_sc as plsc

N = 10000
E = 320000
D_IN = 128
H = 32
OUT = 32
G = 64

NC, NS = 2, 16
NW = NC * NS
NPAD = 10240
EPAD = 327680
EPW = EPAD // NW
SLAB = 128
SLABS_PW = EPW // SLAB
CS = 8
NCHUNK = SLABS_PW // CS
RPT = NPAD // NS
BM = 1024
GRID = NPAD // BM

_sc_mesh = plsc.VectorSubcoreMesh(
    core_axis_name="c", subcore_axis_name="s", num_cores=NC, num_subcores=NS)
_sc_params = pltpu.CompilerParams(use_tc_tiling_on_sc=False)


@functools.partial(
    pl.kernel,
    out_type=jax.ShapeDtypeStruct((NC, NPAD), jnp.float32),
    mesh=_sc_mesh,
    compiler_params=_sc_params,
    scratch_types=[
        pltpu.VMEM_SHARED((NPAD,), jnp.float32),
        pltpu.VMEM((CS, SLAB), jnp.int32),
        pltpu.VMEM((SLAB,), jnp.float32),
        pltpu.VMEM((RPT,), jnp.float32),
    ],
)
def _deg_sc(dst_hbm, out_hbm, acc, idx_v, ones_v, buf_v):
    c = lax.axis_index("c")
    s = lax.axis_index("s")
    wid = s * NC + c

    def _zero(i, _):
        buf_v[pl.ds(i * 16, 16)] = jnp.zeros((16,), jnp.float32)
        return 0
    lax.fori_loop(0, RPT // 16, _zero, 0)
    pltpu.sync_copy(buf_v, acc.at[pl.ds(s * RPT, RPT)])

    def _ones(i, _):
        ones_v[pl.ds(i * 16, 16)] = jnp.ones((16,), jnp.float32)
        return 0
    lax.fori_loop(0, SLAB // 16, _ones, 0)
    plsc.subcore_barrier()

    def _chunk(g, _):
        slab0 = wid * SLABS_PW + g * CS
        pltpu.sync_copy(dst_hbm.at[pl.ds(slab0, CS)], idx_v)
        for j in range(CS):
            pltpu.sync_copy(ones_v, acc.at[idx_v.at[j]], add=True)
        return 0
    lax.fori_loop(0, NCHUNK, _chunk, 0)

    plsc.subcore_barrier()
    pltpu.sync_copy(acc.at[pl.ds(s * RPT, RPT)], buf_v)
    pltpu.sync_copy(buf_v, out_hbm.at[c, pl.ds(s * RPT, RPT)])


@functools.partial(
    pl.kernel,
    out_type=jax.ShapeDtypeStruct((NC, NPAD, H), jnp.float32),
    mesh=_sc_mesh,
    compiler_params=_sc_params,
    scratch_types=[
        pltpu.VMEM_SHARED((NPAD, H), jnp.float32),
        pltpu.VMEM_SHARED((NPAD, H), jnp.float32),
        pltpu.VMEM((CS, SLAB), jnp.int32),
        pltpu.VMEM((CS, SLAB), jnp.int32),
        pltpu.VMEM((CS * SLAB, H), jnp.float32),
        pltpu.SemaphoreType.DMA,
        pltpu.SemaphoreType.DMA,
    ],
)
def _prop_sc(y_hbm, src_hbm, dst_hbm, out_hbm, acc, y_s, idx_s, idx_d, rows_v,
             sem, sem_s):
    c = lax.axis_index("c")
    s = lax.axis_index("s")
    wid = s * NC + c
    rbase = s * RPT

    pltpu.sync_copy(y_hbm.at[pl.ds(rbase, RPT)], rows_v.at[pl.ds(0, RPT)])
    pltpu.sync_copy(rows_v.at[pl.ds(0, RPT)], y_s.at[pl.ds(rbase, RPT)])

    @pl.when(c == 0)
    def _():
        pltpu.sync_copy(rows_v.at[pl.ds(0, RPT)], acc.at[pl.ds(rbase, RPT)])

    @pl.when(c != 0)
    def _():
        def _zero(i, _):
            rows_v[i, pl.ds(0, 16)] = jnp.zeros((16,), jnp.float32)
            rows_v[i, pl.ds(16, 16)] = jnp.zeros((16,), jnp.float32)
            return 0
        lax.fori_loop(0, RPT, _zero, 0)
        pltpu.sync_copy(rows_v.at[pl.ds(0, RPT)], acc.at[pl.ds(rbase, RPT)])

    plsc.subcore_barrier()

    def _chunk(g, _):
        slab0 = wid * SLABS_PW + g * CS
        pltpu.sync_copy(src_hbm.at[pl.ds(slab0, CS)], idx_s)
        pltpu.sync_copy(dst_hbm.at[pl.ds(slab0, CS)], idx_d)
        cps = [
            pltpu.async_copy(
                y_s.at[idx_s.at[j]], rows_v.at[pl.ds(j * SLAB, SLAB)], sem)
            for j in range(CS)
        ]
        sps = []
        for j in range(CS):
            cps[j].wait()
            sps.append(pltpu.async_copy(
                rows_v.at[pl.ds(j * SLAB, SLAB)], acc.at[idx_d.at[j]], sem_s,
                add=True))
        for sp in sps:
            sp.wait()
        return 0
    lax.fori_loop(0, NCHUNK, _chunk, 0)

    plsc.subcore_barrier()
    pltpu.sync_copy(acc.at[pl.ds(rbase, RPT)], rows_v.at[pl.ds(0, RPT)])
    pltpu.sync_copy(rows_v.at[pl.ds(0, RPT)], out_hbm.at[c, pl.ds(rbase, RPT)])


def _k1_body(x_ref, w_ref, degp_ref, y_ref, dinv_ref):
    i = pl.program_id(0)
    deg = degp_ref[0] + degp_ref[1] + 1.0
    rows = i * BM + lax.broadcasted_iota(jnp.int32, (BM, 1), 0)
    dinv = jnp.where(rows < N, lax.rsqrt(jnp.maximum(deg, 1.0)), 0.0)
    xw = jnp.dot(x_ref[...], w_ref[...], preferred_element_type=jnp.float32)
    y_ref[...] = xw * dinv
    dinv_ref[...] = dinv


def _k1_call(xp, W1, degp3):
    return pl.pallas_call(
        _k1_body,
        grid=(GRID,),
        in_specs=[
            pl.BlockSpec((BM, D_IN), lambda i: (i, 0)),
            pl.BlockSpec((D_IN, H), lambda i: (0, 0)),
            pl.BlockSpec((NC, BM, 1), lambda i: (0, i, 0)),
        ],
        out_specs=[
            pl.BlockSpec((BM, H), lambda i: (i, 0)),
            pl.BlockSpec((BM, 1), lambda i: (i, 0)),
        ],
        out_shape=[
            jax.ShapeDtypeStruct((NPAD, H), jnp.float32),
            jax.ShapeDtypeStruct((NPAD, 1), jnp.float32),
        ],
    )(xp, W1, degp3)


def _bnrc_body(p_ref, dinv_ref, gb_ref, w_ref, y_ref, st_v):
    i = pl.program_id(0)
    dinv = dinv_ref[...]
    t = (p_ref[0] + p_ref[1]) * dinv

    @pl.when(i == 0)
    def _():
        st_v[...] = jnp.zeros_like(st_v)

    @pl.when(i < GRID)
    def _():
        st_v[...] += jnp.concatenate(
            [jnp.sum(t, axis=0, keepdims=True),
             jnp.sum(t * t, axis=0, keepdims=True)], axis=0)

    @pl.when(i >= GRID)
    def _():
        m = st_v[0:1, :] * (1.0 / N)
        v = st_v[1:2, :] * (1.0 / N) - m * m
        a = gb_ref[0:1, :] * lax.rsqrt(v + 1e-5)
        c2 = gb_ref[1:2, :] - m * a
        z = jnp.maximum(t * a + c2, 0.0)
        y_ref[...] = jnp.dot(
            z, w_ref[...], preferred_element_type=jnp.float32) * dinv


def _bnrc_call(p, dinv, gb, W):
    return pl.pallas_call(
        _bnrc_body,
        grid=(2 * GRID,),
        in_specs=[
            pl.BlockSpec((NC, BM, H), lambda i: (0, i % GRID, 0)),
            pl.BlockSpec((BM, 1), lambda i: (i % GRID, 0)),
            pl.BlockSpec((2, H), lambda i: (0, 0)),
            pl.BlockSpec((H, H), lambda i: (0, 0)),
        ],
        out_specs=pl.BlockSpec((BM, H), lambda i: (i % GRID, 0)),
        out_shape=jax.ShapeDtypeStruct((NPAD, H), jnp.float32),
        scratch_shapes=[pltpu.VMEM((2, H), jnp.float32)],
    )(p, dinv, gb, W)


def _pool_body(p_ref, dinv_ref, b3_ref, batch_ref, out_ref, sums_v, cnt_v):
    i = pl.program_id(0)

    @pl.when(i == 0)
    def _():
        sums_v[...] = jnp.zeros_like(sums_v)
        cnt_v[...] = jnp.zeros_like(cnt_v)

    t = (p_ref[0] + p_ref[1]) * dinv_ref[...]
    h = t + b3_ref[...]
    b = batch_ref[0]
    oh = (lax.broadcasted_iota(jnp.int32, (G, BM), 0) == b
          ).astype(jnp.float32)
    sums_v[...] += jnp.dot(oh, h, preferred_element_type=jnp.float32)
    cnt_v[...] += jnp.sum(oh, axis=1, keepdims=True)
    out_ref[...] = sums_v[...] / jnp.maximum(cnt_v[...], 1.0)


def _pool_call(p, dinv, b3, batchp):
    return pl.pallas_call(
        _pool_body,
        grid=(GRID,),
        in_specs=[
            pl.BlockSpec((NC, BM, H), lambda i: (0, i, 0)),
            pl.BlockSpec((BM, 1), lambda i: (i, 0)),
            pl.BlockSpec((1, H), lambda i: (0, 0)),
            pl.BlockSpec((1, 1, BM), lambda i: (i, 0, 0)),
        ],
        out_specs=pl.BlockSpec((G, OUT), lambda i: (0, 0)),
        out_shape=jax.ShapeDtypeStruct((G, OUT), jnp.float32),
        scratch_shapes=[
            pltpu.VMEM((G, OUT), jnp.float32),
            pltpu.VMEM((G, 1), jnp.float32),
        ],
    )(p, dinv, b3, batchp)


def kernel(x, edge_index, batch, W1, b1, W2, b2, W3, b3,
           gamma1, beta1, gamma2, beta2):
    x = x.astype(jnp.float32)
    src = edge_index[0].astype(jnp.int32)
    dst = edge_index[1].astype(jnp.int32)
    srcp = jnp.concatenate(
        [src, jnp.full((EPAD - E,), N, jnp.int32)]).reshape(EPAD // SLAB, SLAB)
    dstp = jnp.concatenate(
        [dst, jnp.full((EPAD - E,), NPAD - 1, jnp.int32)]
    ).reshape(EPAD // SLAB, SLAB)
    xp = jnp.pad(x, ((0, NPAD - N), (0, 0)))
    batchp = jnp.concatenate(
        [batch.astype(jnp.int32), jnp.full((NPAD - N,), G, jnp.int32)]
    ).reshape(GRID, 1, BM)

    degp = _deg_sc(dstp)
    y1, dinv = _k1_call(xp, W1, degp.reshape(NC, NPAD, 1))

    p1 = _prop_sc(y1, srcp, dstp)
    y2 = _bnrc_call(p1, dinv, jnp.stack([gamma1, beta1]), W2)

    p2 = _prop_sc(y2, srcp, dstp)
    y3 = _bnrc_call(p2, dinv, jnp.stack([gamma2, beta2]), W3)

    p3 = _prop_sc(y3, srcp, dstp)
    return _pool_call(p3, dinv, b3.reshape(1, OUT), batchp)


# speedup vs baseline: 1.7758x; 1.0255x over previous
import functools

import jax
import jax.numpy as jnp
from jax import lax
from jax.experimental import pallas as pl
from jax.experimental.pallas import tpu as pltpu
from jax.experimental.pallas import tpu_sc as plsc

N = 10000
E = 320000
D_IN = 128
H = 32
OUT = 32
G = 64

NC, NS = 2, 16
NW = NC * NS
NPAD = 10240
EPAD = 327680
EPW = EPAD // NW
SLAB = 128
SLABS_PW = EPW // SLAB
CS = 8
NCHUNK = SLABS_PW // CS
CP = 5
NB = SLABS_PW // CP // 2
RPT = NPAD // NS
BM = 1024
GRID = NPAD // BM

_sc_mesh = plsc.VectorSubcoreMesh(
    core_axis_name="c", subcore_axis_name="s", num_cores=NC, num_subcores=NS)
_sc_params = pltpu.CompilerParams(use_tc_tiling_on_sc=False)


@functools.partial(
    pl.kernel,
    out_type=jax.ShapeDtypeStruct((NC, NPAD), jnp.float32),
    mesh=_sc_mesh,
    compiler_params=_sc_params,
    scratch_types=[
        pltpu.VMEM_SHARED((NPAD,), jnp.float32),
        pltpu.VMEM((CS, SLAB), jnp.int32),
        pltpu.VMEM((SLAB,), jnp.float32),
        pltpu.VMEM((RPT,), jnp.float32),
    ],
)
def _deg_sc(dst_hbm, out_hbm, acc, idx_v, ones_v, buf_v):
    c = lax.axis_index("c")
    s = lax.axis_index("s")
    wid = s * NC + c

    def _zero(i, _):
        buf_v[pl.ds(i * 16, 16)] = jnp.zeros((16,), jnp.float32)
        return 0
    lax.fori_loop(0, RPT // 16, _zero, 0)
    pltpu.sync_copy(buf_v, acc.at[pl.ds(s * RPT, RPT)])

    def _ones(i, _):
        ones_v[pl.ds(i * 16, 16)] = jnp.ones((16,), jnp.float32)
        return 0
    lax.fori_loop(0, SLAB // 16, _ones, 0)
    plsc.subcore_barrier()

    def _chunk(g, _):
        slab0 = wid * SLABS_PW + g * CS
        pltpu.sync_copy(dst_hbm.at[pl.ds(slab0, CS)], idx_v)
        for j in range(CS):
            pltpu.sync_copy(ones_v, acc.at[idx_v.at[j]], add=True)
        return 0
    lax.fori_loop(0, NCHUNK, _chunk, 0)

    plsc.subcore_barrier()
    pltpu.sync_copy(acc.at[pl.ds(s * RPT, RPT)], buf_v)
    pltpu.sync_copy(buf_v, out_hbm.at[c, pl.ds(s * RPT, RPT)])


@functools.partial(
    pl.kernel,
    out_type=jax.ShapeDtypeStruct((NC, NPAD, H), jnp.float32),
    mesh=_sc_mesh,
    compiler_params=_sc_params,
    scratch_types=[
        pltpu.VMEM_SHARED((NPAD, H), jnp.float32),
        pltpu.VMEM_SHARED((NPAD, H), jnp.float32),
        pltpu.VMEM((CP, SLAB), jnp.int32),
        pltpu.VMEM((CP, SLAB), jnp.int32),
        pltpu.VMEM((CP, SLAB), jnp.int32),
        pltpu.VMEM((CP, SLAB), jnp.int32),
        pltpu.VMEM((CP * SLAB, H), jnp.float32),
        pltpu.VMEM((CP * SLAB, H), jnp.float32),
        pltpu.SemaphoreType.DMA,
        pltpu.SemaphoreType.DMA,
    ],
)
def _prop_sc(y_hbm, src_hbm, dst_hbm, out_hbm, acc, y_s, iA_s, iA_d, iB_s,
             iB_d, rowsA, rowsB, sem_g, sem_s):
    c = lax.axis_index("c")
    s = lax.axis_index("s")
    wid = s * NC + c
    rbase = s * RPT

    pltpu.sync_copy(y_hbm.at[pl.ds(rbase, RPT)], rowsA)
    pltpu.sync_copy(rowsA, y_s.at[pl.ds(rbase, RPT)])

    @pl.when(c == 0)
    def _():
        pltpu.sync_copy(rowsA, acc.at[pl.ds(rbase, RPT)])

    @pl.when(c != 0)
    def _():
        def _zero(i, _):
            rowsA[i, pl.ds(0, 16)] = jnp.zeros((16,), jnp.float32)
            rowsA[i, pl.ds(16, 16)] = jnp.zeros((16,), jnp.float32)
            return 0
        lax.fori_loop(0, RPT, _zero, 0)
        pltpu.sync_copy(rowsA, acc.at[pl.ds(rbase, RPT)])

    plsc.subcore_barrier()

    CB = CP * SLAB * H * 4

    def _drainA():
        pltpu.make_async_copy(y_hbm.at[pl.ds(0, CP * SLAB)], rowsA, sem_s).wait()

    def _drainB():
        pltpu.make_async_copy(y_hbm.at[pl.ds(0, CP * SLAB)], rowsB, sem_s).wait()

    def _body(i, _):
        a0 = wid * SLABS_PW + 2 * i * CP
        b0 = a0 + CP

        @pl.when(i > 0)
        def _():
            _drainA()
        pltpu.sync_copy(src_hbm.at[pl.ds(a0, CP)], iA_s)
        pltpu.sync_copy(dst_hbm.at[pl.ds(a0, CP)], iA_d)
        gA = [pltpu.async_copy(
                  y_s.at[iA_s.at[j]], rowsA.at[pl.ds(j * SLAB, SLAB)], sem_g)
              for j in range(CP)]
        for j in range(CP):
            gA[j].wait()
            pltpu.async_copy(
                rowsA.at[pl.ds(j * SLAB, SLAB)], acc.at[iA_d.at[j]], sem_s,
                add=True)

        @pl.when(i > 0)
        def _():
            _drainB()
        pltpu.sync_copy(src_hbm.at[pl.ds(b0, CP)], iB_s)
        pltpu.sync_copy(dst_hbm.at[pl.ds(b0, CP)], iB_d)
        gB = [pltpu.async_copy(
                  y_s.at[iB_s.at[j]], rowsB.at[pl.ds(j * SLAB, SLAB)], sem_g)
              for j in range(CP)]
        for j in range(CP):
            gB[j].wait()
            pltpu.async_copy(
                rowsB.at[pl.ds(j * SLAB, SLAB)], acc.at[iB_d.at[j]], sem_s,
                add=True)
        return 0
    lax.fori_loop(0, NB, _body, 0)
    _drainA()
    _drainB()

    plsc.subcore_barrier()
    pltpu.sync_copy(acc.at[pl.ds(rbase, RPT)], rowsA)
    pltpu.sync_copy(rowsA, out_hbm.at[c, pl.ds(rbase, RPT)])


def _k1_body(x_ref, w_ref, degp_ref, y_ref, dinv_ref):
    i = pl.program_id(0)
    deg = degp_ref[0] + degp_ref[1] + 1.0
    rows = i * BM + lax.broadcasted_iota(jnp.int32, (BM, 1), 0)
    dinv = jnp.where(rows < N, lax.rsqrt(jnp.maximum(deg, 1.0)), 0.0)
    xw = jnp.dot(x_ref[...], w_ref[...], preferred_element_type=jnp.float32)
    y_ref[...] = xw * dinv
    dinv_ref[...] = dinv


def _k1_call(xp, W1, degp3):
    return pl.pallas_call(
        _k1_body,
        grid=(GRID,),
        in_specs=[
            pl.BlockSpec((BM, D_IN), lambda i: (i, 0)),
            pl.BlockSpec((D_IN, H), lambda i: (0, 0)),
            pl.BlockSpec((NC, BM, 1), lambda i: (0, i, 0)),
        ],
        out_specs=[
            pl.BlockSpec((BM, H), lambda i: (i, 0)),
            pl.BlockSpec((BM, 1), lambda i: (i, 0)),
        ],
        out_shape=[
            jax.ShapeDtypeStruct((NPAD, H), jnp.float32),
            jax.ShapeDtypeStruct((NPAD, 1), jnp.float32),
        ],
    )(xp, W1, degp3)


def _bnrc_body(p_ref, dinv_ref, gb_ref, w_ref, y_ref, st_v):
    i = pl.program_id(0)
    dinv = dinv_ref[...]
    t = (p_ref[0] + p_ref[1]) * dinv

    @pl.when(i == 0)
    def _():
        st_v[...] = jnp.zeros_like(st_v)

    @pl.when(i < GRID)
    def _():
        st_v[...] += jnp.concatenate(
            [jnp.sum(t, axis=0, keepdims=True),
             jnp.sum(t * t, axis=0, keepdims=True)], axis=0)

    @pl.when(i >= GRID)
    def _():
        m = st_v[0:1, :] * (1.0 / N)
        v = st_v[1:2, :] * (1.0 / N) - m * m
        a = gb_ref[0:1, :] * lax.rsqrt(v + 1e-5)
        c2 = gb_ref[1:2, :] - m * a
        z = jnp.maximum(t * a + c2, 0.0)
        y_ref[...] = jnp.dot(
            z, w_ref[...], preferred_element_type=jnp.float32) * dinv


def _bnrc_call(p, dinv, gb, W):
    return pl.pallas_call(
        _bnrc_body,
        grid=(2 * GRID,),
        in_specs=[
            pl.BlockSpec((NC, BM, H), lambda i: (0, i % GRID, 0)),
            pl.BlockSpec((BM, 1), lambda i: (i % GRID, 0)),
            pl.BlockSpec((2, H), lambda i: (0, 0)),
            pl.BlockSpec((H, H), lambda i: (0, 0)),
        ],
        out_specs=pl.BlockSpec((BM, H), lambda i: (i % GRID, 0)),
        out_shape=jax.ShapeDtypeStruct((NPAD, H), jnp.float32),
        scratch_shapes=[pltpu.VMEM((2, H), jnp.float32)],
    )(p, dinv, gb, W)


def _pool_body(p_ref, dinv_ref, b3_ref, batch_ref, out_ref, sums_v, cnt_v):
    i = pl.program_id(0)

    @pl.when(i == 0)
    def _():
        sums_v[...] = jnp.zeros_like(sums_v)
        cnt_v[...] = jnp.zeros_like(cnt_v)

    t = (p_ref[0] + p_ref[1]) * dinv_ref[...]
    h = t + b3_ref[...]
    b = batch_ref[0]
    oh = (lax.broadcasted_iota(jnp.int32, (G, BM), 0) == b
          ).astype(jnp.float32)
    sums_v[...] += jnp.dot(oh, h, preferred_element_type=jnp.float32)
    cnt_v[...] += jnp.sum(oh, axis=1, keepdims=True)
    out_ref[...] = sums_v[...] / jnp.maximum(cnt_v[...], 1.0)


def _pool_call(p, dinv, b3, batchp):
    return pl.pallas_call(
        _pool_body,
        grid=(GRID,),
        in_specs=[
            pl.BlockSpec((NC, BM, H), lambda i: (0, i, 0)),
            pl.BlockSpec((BM, 1), lambda i: (i, 0)),
            pl.BlockSpec((1, H), lambda i: (0, 0)),
            pl.BlockSpec((1, 1, BM), lambda i: (i, 0, 0)),
        ],
        out_specs=pl.BlockSpec((G, OUT), lambda i: (0, 0)),
        out_shape=jax.ShapeDtypeStruct((G, OUT), jnp.float32),
        scratch_shapes=[
            pltpu.VMEM((G, OUT), jnp.float32),
            pltpu.VMEM((G, 1), jnp.float32),
        ],
    )(p, dinv, b3, batchp)


def kernel(x, edge_index, batch, W1, b1, W2, b2, W3, b3,
           gamma1, beta1, gamma2, beta2):
    x = x.astype(jnp.float32)
    src = edge_index[0].astype(jnp.int32)
    dst = edge_index[1].astype(jnp.int32)
    srcp = jnp.concatenate(
        [src, jnp.full((EPAD - E,), N, jnp.int32)]).reshape(EPAD // SLAB, SLAB)
    dstp = jnp.concatenate(
        [dst, jnp.full((EPAD - E,), NPAD - 1, jnp.int32)]
    ).reshape(EPAD // SLAB, SLAB)
    xp = jnp.pad(x, ((0, NPAD - N), (0, 0)))
    batchp = jnp.concatenate(
        [batch.astype(jnp.int32), jnp.full((NPAD - N,), G, jnp.int32)]
    ).reshape(GRID, 1, BM)

    degp = _deg_sc(dstp)
    y1, dinv = _k1_call(xp, W1, degp.reshape(NC, NPAD, 1))

    p1 = _prop_sc(y1, srcp, dstp)
    y2 = _bnrc_call(p1, dinv, jnp.stack([gamma1, beta1]), W2)

    p2 = _prop_sc(y2, srcp, dstp)
    y3 = _bnrc_call(p2, dinv, jnp.stack([gamma2, beta2]), W3)

    p3 = _prop_sc(y3, srcp, dstp)
    return _pool_call(p3, dinv, b3.reshape(1, OUT), batchp)


# speedup vs baseline: 1.8967x; 1.0681x over previous
import functools

import jax
import jax.numpy as jnp
from jax import lax
from jax.experimental import pallas as pl
from jax.experimental.pallas import tpu as pltpu
from jax.experimental.pallas import tpu_sc as plsc

N = 10000
E = 320000
D_IN = 128
H = 32
OUT = 32
G = 64

NC, NS = 2, 16
NW = NC * NS
NPAD = 10240
EPAD = 327680
EPW = EPAD // NW
SLAB = 128
SLABS_PW = EPW // SLAB
CS = 8
NCHUNK = SLABS_PW // CS
CP = 5
NB = SLABS_PW // CP // 2
RPT = NPAD // NS
BM = 1024
GRID = NPAD // BM

_sc_mesh = plsc.VectorSubcoreMesh(
    core_axis_name="c", subcore_axis_name="s", num_cores=NC, num_subcores=NS)
_sc_params = pltpu.CompilerParams(use_tc_tiling_on_sc=False)


@functools.partial(
    pl.kernel,
    out_type=jax.ShapeDtypeStruct((NC, NPAD), jnp.float32),
    mesh=_sc_mesh,
    compiler_params=_sc_params,
    scratch_types=[
        pltpu.VMEM_SHARED((NPAD,), jnp.float32),
        pltpu.VMEM((CS, SLAB), jnp.int32),
        pltpu.VMEM((SLAB,), jnp.float32),
        pltpu.VMEM((RPT,), jnp.float32),
    ],
)
def _deg_sc(ei_hbm, out_hbm, acc, idx_v, ones_v, buf_v):
    c = lax.axis_index("c")
    s = lax.axis_index("s")
    wid = s * NC + c

    def _zero(i, _):
        buf_v[pl.ds(i * 16, 16)] = jnp.zeros((16,), jnp.float32)
        return 0
    lax.fori_loop(0, RPT // 16, _zero, 0)
    pltpu.sync_copy(buf_v, acc.at[pl.ds(s * RPT, RPT)])

    def _ones(i, _):
        ones_v[pl.ds(i * 16, 16)] = jnp.ones((16,), jnp.float32)
        return 0
    lax.fori_loop(0, SLAB // 16, _ones, 0)
    plsc.subcore_barrier()

    def _chunk(g, _):
        slab0 = wid * SLABS_PW + g * CS
        pltpu.sync_copy(ei_hbm.at[1, pl.ds(slab0, CS)], idx_v)
        for j in range(CS):
            pltpu.sync_copy(ones_v, acc.at[idx_v.at[j]], add=True)
        return 0
    lax.fori_loop(0, NCHUNK, _chunk, 0)

    plsc.subcore_barrier()
    pltpu.sync_copy(acc.at[pl.ds(s * RPT, RPT)], buf_v)
    pltpu.sync_copy(buf_v, out_hbm.at[c, pl.ds(s * RPT, RPT)])


@functools.partial(
    pl.kernel,
    out_type=jax.ShapeDtypeStruct((NC, NPAD, H), jnp.float32),
    mesh=_sc_mesh,
    compiler_params=_sc_params,
    scratch_types=[
        pltpu.VMEM_SHARED((NPAD, H), jnp.float32),
        pltpu.VMEM_SHARED((NPAD, H), jnp.float32),
        pltpu.VMEM((CP, SLAB), jnp.int32),
        pltpu.VMEM((CP, SLAB), jnp.int32),
        pltpu.VMEM((CP, SLAB), jnp.int32),
        pltpu.VMEM((CP, SLAB), jnp.int32),
        pltpu.VMEM((CP * SLAB, H), jnp.float32),
        pltpu.VMEM((CP * SLAB, H), jnp.float32),
        pltpu.SemaphoreType.DMA,
        pltpu.SemaphoreType.DMA,
    ],
)
def _prop_sc(y_hbm, ei_hbm, out_hbm, acc, y_s, iA_s, iA_d, iB_s,
             iB_d, rowsA, rowsB, sem_g, sem_s):
    c = lax.axis_index("c")
    s = lax.axis_index("s")
    wid = s * NC + c
    rbase = s * RPT

    pltpu.sync_copy(y_hbm.at[pl.ds(rbase, RPT)], rowsA)
    pltpu.sync_copy(rowsA, y_s.at[pl.ds(rbase, RPT)])

    @pl.when(c == 0)
    def _():
        pltpu.sync_copy(rowsA, acc.at[pl.ds(rbase, RPT)])

    @pl.when(c != 0)
    def _():
        def _zero(i, _):
            rowsA[i, pl.ds(0, 16)] = jnp.zeros((16,), jnp.float32)
            rowsA[i, pl.ds(16, 16)] = jnp.zeros((16,), jnp.float32)
            return 0
        lax.fori_loop(0, RPT, _zero, 0)
        pltpu.sync_copy(rowsA, acc.at[pl.ds(rbase, RPT)])

    plsc.subcore_barrier()

    CB = CP * SLAB * H * 4

    def _drainA():
        pltpu.make_async_copy(y_hbm.at[pl.ds(0, CP * SLAB)], rowsA, sem_s).wait()

    def _drainB():
        pltpu.make_async_copy(y_hbm.at[pl.ds(0, CP * SLAB)], rowsB, sem_s).wait()

    def _body(i, _):
        a0 = wid * SLABS_PW + 2 * i * CP
        b0 = a0 + CP

        @pl.when(i > 0)
        def _():
            _drainA()
        pltpu.sync_copy(ei_hbm.at[0, pl.ds(a0, CP)], iA_s)
        pltpu.sync_copy(ei_hbm.at[1, pl.ds(a0, CP)], iA_d)
        gA = [pltpu.async_copy(
                  y_s.at[iA_s.at[j]], rowsA.at[pl.ds(j * SLAB, SLAB)], sem_g)
              for j in range(CP)]
        for j in range(CP):
            gA[j].wait()
            pltpu.async_copy(
                rowsA.at[pl.ds(j * SLAB, SLAB)], acc.at[iA_d.at[j]], sem_s,
                add=True)

        @pl.when(i > 0)
        def _():
            _drainB()
        pltpu.sync_copy(ei_hbm.at[0, pl.ds(b0, CP)], iB_s)
        pltpu.sync_copy(ei_hbm.at[1, pl.ds(b0, CP)], iB_d)
        gB = [pltpu.async_copy(
                  y_s.at[iB_s.at[j]], rowsB.at[pl.ds(j * SLAB, SLAB)], sem_g)
              for j in range(CP)]
        for j in range(CP):
            gB[j].wait()
            pltpu.async_copy(
                rowsB.at[pl.ds(j * SLAB, SLAB)], acc.at[iB_d.at[j]], sem_s,
                add=True)
        return 0
    lax.fori_loop(0, NB, _body, 0)
    _drainA()
    _drainB()

    plsc.subcore_barrier()
    pltpu.sync_copy(acc.at[pl.ds(rbase, RPT)], rowsA)
    pltpu.sync_copy(rowsA, out_hbm.at[c, pl.ds(rbase, RPT)])


def _k1_body(x_ref, w_ref, degp_ref, y_ref, dinv_ref):
    i = pl.program_id(0)
    deg = degp_ref[0:1, :] + degp_ref[1:2, :] + 1.0
    cols = i * BM + lax.broadcasted_iota(jnp.int32, (1, BM), 1)
    dinv_l = jnp.where(cols < N, lax.rsqrt(jnp.maximum(deg, 1.0)), 0.0)
    dinv = dinv_l.reshape(BM, 1)
    xw = jnp.dot(x_ref[...], w_ref[...], preferred_element_type=jnp.float32)
    y_ref[...] = xw * dinv
    dinv_ref[...] = dinv


def _k1_call(xp, W1, degp):
    return pl.pallas_call(
        _k1_body,
        grid=(GRID,),
        in_specs=[
            pl.BlockSpec((BM, D_IN), lambda i: (i, 0)),
            pl.BlockSpec((D_IN, H), lambda i: (0, 0)),
            pl.BlockSpec((NC, BM), lambda i: (0, i)),
        ],
        out_specs=[
            pl.BlockSpec((BM, H), lambda i: (i, 0)),
            pl.BlockSpec((BM, 1), lambda i: (i, 0)),
        ],
        out_shape=[
            jax.ShapeDtypeStruct((NPAD, H), jnp.float32),
            jax.ShapeDtypeStruct((NPAD, 1), jnp.float32),
        ],
    )(xp, W1, degp)


def _bnrc_body(p_ref, dinv_ref, gb_ref, w_ref, y_ref, st_v):
    i = pl.program_id(0)
    dinv = dinv_ref[...]
    t = (p_ref[0] + p_ref[1]) * dinv

    @pl.when(i == 0)
    def _():
        st_v[...] = jnp.zeros_like(st_v)

    @pl.when(i < GRID)
    def _():
        st_v[...] += jnp.concatenate(
            [jnp.sum(t, axis=0, keepdims=True),
             jnp.sum(t * t, axis=0, keepdims=True)], axis=0)

    @pl.when(i >= GRID)
    def _():
        m = st_v[0:1, :] * (1.0 / N)
        v = st_v[1:2, :] * (1.0 / N) - m * m
        a = gb_ref[0:1, :] * lax.rsqrt(v + 1e-5)
        c2 = gb_ref[1:2, :] - m * a
        z = jnp.maximum(t * a + c2, 0.0)
        y_ref[...] = jnp.dot(
            z, w_ref[...], preferred_element_type=jnp.float32) * dinv


def _bnrc_call(p, dinv, gb, W):
    return pl.pallas_call(
        _bnrc_body,
        grid=(2 * GRID,),
        in_specs=[
            pl.BlockSpec((NC, BM, H), lambda i: (0, i % GRID, 0)),
            pl.BlockSpec((BM, 1), lambda i: (i % GRID, 0)),
            pl.BlockSpec((2, H), lambda i: (0, 0)),
            pl.BlockSpec((H, H), lambda i: (0, 0)),
        ],
        out_specs=pl.BlockSpec((BM, H), lambda i: (i % GRID, 0)),
        out_shape=jax.ShapeDtypeStruct((NPAD, H), jnp.float32),
        scratch_shapes=[pltpu.VMEM((2, H), jnp.float32)],
    )(p, dinv, gb, W)


def _pool_body(p_ref, dinv_ref, b3_ref, batch_ref, out_ref, sums_v, cnt_v):
    i = pl.program_id(0)

    @pl.when(i == 0)
    def _():
        sums_v[...] = jnp.zeros_like(sums_v)
        cnt_v[...] = jnp.zeros_like(cnt_v)

    t = (p_ref[0] + p_ref[1]) * dinv_ref[...]
    h = t + b3_ref[...]
    b = batch_ref[0]
    oh = (lax.broadcasted_iota(jnp.int32, (G, BM), 0) == b
          ).astype(jnp.float32)
    sums_v[...] += jnp.dot(oh, h, preferred_element_type=jnp.float32)
    cnt_v[...] += jnp.sum(oh, axis=1, keepdims=True)
    out_ref[...] = sums_v[...] / jnp.maximum(cnt_v[...], 1.0)


def _pool_call(p, dinv, b3, batchp):
    return pl.pallas_call(
        _pool_body,
        grid=(GRID,),
        in_specs=[
            pl.BlockSpec((NC, BM, H), lambda i: (0, i, 0)),
            pl.BlockSpec((BM, 1), lambda i: (i, 0)),
            pl.BlockSpec((1, H), lambda i: (0, 0)),
            pl.BlockSpec((1, 1, BM), lambda i: (i, 0, 0)),
        ],
        out_specs=pl.BlockSpec((G, OUT), lambda i: (0, 0)),
        out_shape=jax.ShapeDtypeStruct((G, OUT), jnp.float32),
        scratch_shapes=[
            pltpu.VMEM((G, OUT), jnp.float32),
            pltpu.VMEM((G, 1), jnp.float32),
        ],
    )(p, dinv, b3, batchp)


def kernel(x, edge_index, batch, W1, b1, W2, b2, W3, b3,
           gamma1, beta1, gamma2, beta2):
    x = x.astype(jnp.float32)
    eip = jnp.concatenate(
        [edge_index.astype(jnp.int32),
         jnp.stack([jnp.full((EPAD - E,), N, jnp.int32),
                    jnp.full((EPAD - E,), NPAD - 1, jnp.int32)])],
        axis=1).reshape(2, EPAD // SLAB, SLAB)
    xp = jnp.pad(x, ((0, NPAD - N), (0, 0)))
    batchp = jnp.concatenate(
        [batch.astype(jnp.int32), jnp.full((NPAD - N,), G, jnp.int32)]
    ).reshape(GRID, 1, BM)

    degp = _deg_sc(eip)
    y1, dinv = _k1_call(xp, W1, degp)

    p1 = _prop_sc(y1, eip)
    y2 = _bnrc_call(p1, dinv, jnp.stack([gamma1, beta1]), W2)

    p2 = _prop_sc(y2, eip)
    y3 = _bnrc_call(p2, dinv, jnp.stack([gamma2, beta2]), W3)

    p3 = _prop_sc(y3, eip)
    return _pool_call(p3, dinv, b3.reshape(1, OUT), batchp)
